# Initial kernel scaffold; baseline (speedup 1.0000x reference)
import jax
import jax.numpy as jnp
from jax import lax
from jax.experimental import pallas as pl
from jax.experimental.pallas import tpu as pltpu
from jax.experimental.pallas import tpu_sc as plsc

BATCH = 16384
HIST = 50
DIM = 64
NROWS = BATCH * HIST
CHUNK = 128
NCHUNKS = NROWS // CHUNK
NW = 32
CPW = NCHUNKS // NW


def _embed_kernel(table_hbm, idx_hbm, out_hbm, idx_v, rows_v, gsem):
  nc = 2
  wid = lax.axis_index("s") * nc + lax.axis_index("c")
  base = wid * CPW
  pltpu.sync_copy(idx_hbm.at[pl.ds(base, CPW)], idx_v)

  def body(j, _):
    pltpu.async_copy(table_hbm.at[idx_v.at[j]], rows_v, gsem).wait()
    pltpu.sync_copy(rows_v, out_hbm.at[pl.ds((base + j) * CHUNK, CHUNK)])
    return 0

  lax.fori_loop(0, CPW, body, 0)


@jax.jit
def _embed(table, idx2d):
  mesh = plsc.VectorSubcoreMesh(core_axis_name="c", subcore_axis_name="s")
  return pl.kernel(
      _embed_kernel,
      out_type=jax.ShapeDtypeStruct((NROWS, DIM), jnp.float32),
      mesh=mesh,
      scratch_types=[
          pltpu.VMEM((CPW, CHUNK), jnp.int32),
          pltpu.VMEM((CHUNK, DIM), jnp.float32),
          pltpu.SemaphoreType.DMA,
      ],
  )(table, idx2d)


def kernel(x, table):
  idx2d = x.reshape(NCHUNKS, CHUNK)
  out = _embed(table, idx2d)
  return out.reshape(BATCH, HIST, DIM)


# speedup vs baseline: 1.6829x; 1.6829x over previous
import jax
import jax.numpy as jnp
from jax import lax
from jax.experimental import pallas as pl
from jax.experimental.pallas import tpu as pltpu
from jax.experimental.pallas import tpu_sc as plsc

BATCH = 16384
HIST = 50
DIM = 64
NROWS = BATCH * HIST
CHUNK = 128
NCHUNKS = NROWS // CHUNK
NW = 32
CPW = NCHUNKS // NW


def _embed_kernel(table_hbm, idx_hbm, out_hbm, idx_v, rows_v, gsem):
  nc = 2
  wid = lax.axis_index("s") * nc + lax.axis_index("c")
  base = wid * CPW
  pltpu.sync_copy(idx_hbm.at[pl.ds(base, CPW)], idx_v)

  def body(j, _):
    pltpu.async_copy(table_hbm.at[idx_v.at[j]], rows_v, gsem).wait()
    pltpu.sync_copy(rows_v, out_hbm.at[pl.ds((base + j) * CHUNK, CHUNK)])
    return 0

  lax.fori_loop(0, CPW, body, 0)


@jax.jit
def _embed(table, idx2d):
  mesh = plsc.VectorSubcoreMesh(core_axis_name="c", subcore_axis_name="s")
  return pl.kernel(
      _embed_kernel,
      out_type=jax.ShapeDtypeStruct((NROWS, DIM), jnp.float32),
      mesh=mesh,
      scratch_types=[
          pltpu.VMEM((CPW, CHUNK), jnp.int32),
          pltpu.VMEM((CHUNK, DIM), jnp.float32),
          pltpu.SemaphoreType.DMA,
      ],
      compiler_params=pltpu.CompilerParams(use_tc_tiling_on_sc=False),
  )(table, idx2d)


def kernel(x, table):
  idx2d = x.reshape(NCHUNKS, CHUNK)
  out = _embed(table, idx2d)
  return out.reshape(BATCH, HIST, DIM)


# speedup vs baseline: 1.8724x; 1.1126x over previous
import jax
import jax.numpy as jnp
from jax import lax
from jax.experimental import pallas as pl
from jax.experimental.pallas import tpu as pltpu
from jax.experimental.pallas import tpu_sc as plsc

BATCH = 16384
HIST = 50
DIM = 64
NROWS = BATCH * HIST
CHUNK = 128
NCHUNKS = NROWS // CHUNK
NW = 32
CPW = NCHUNKS // NW
NBUF = 8
NRING = CPW // NBUF


def _embed_kernel(table_hbm, idx_hbm, out_hbm, idx_v, *rest):
  rows = rest[:NBUF]
  gsem = rest[NBUF:2 * NBUF]
  wsem = rest[2 * NBUF:]
  nc = 2
  wid = lax.axis_index("s") * nc + lax.axis_index("c")
  base = wid * CPW
  pltpu.sync_copy(idx_hbm.at[pl.ds(base, CPW)], idx_v)

  for b in range(NBUF):
    pltpu.async_copy(table_hbm.at[idx_v.at[b]], rows[b], gsem[b])

  @pl.loop(0, NRING)
  def _ring(r):
    j0 = r * NBUF
    for b in range(NBUF):
      j = j0 + b
      pltpu.make_async_copy(table_hbm.at[idx_v.at[j]], rows[b],
                            gsem[b]).wait()
      pltpu.async_copy(rows[b],
                       out_hbm.at[pl.ds((base + j) * CHUNK, CHUNK)], wsem[b])
    for b in range(NBUF):
      jn = j0 + NBUF + b

      @pl.when(jn < CPW)
      def _():
        pltpu.make_async_copy(rows[b], out_hbm.at[pl.ds(0, CHUNK)],
                              wsem[b]).wait()
        pltpu.async_copy(table_hbm.at[idx_v.at[jn]], rows[b], gsem[b])

  for b in range(NBUF):
    pltpu.make_async_copy(rows[b], out_hbm.at[pl.ds(0, CHUNK)],
                          wsem[b]).wait()


@jax.jit
def _embed(table, idx2d):
  mesh = plsc.VectorSubcoreMesh(core_axis_name="c", subcore_axis_name="s")
  return pl.kernel(
      _embed_kernel,
      out_type=jax.ShapeDtypeStruct((NROWS, DIM), jnp.float32),
      mesh=mesh,
      scratch_types=(
          [pltpu.VMEM((CPW, CHUNK), jnp.int32)]
          + [pltpu.VMEM((CHUNK, DIM), jnp.float32) for _ in range(NBUF)]
          + [pltpu.SemaphoreType.DMA for _ in range(2 * NBUF)]
      ),
      compiler_params=pltpu.CompilerParams(use_tc_tiling_on_sc=False),
  )(table, idx2d)


def kernel(x, table):
  idx2d = x.reshape(NCHUNKS, CHUNK)
  out = _embed(table, idx2d)
  return out.reshape(BATCH, HIST, DIM)
